# baseline (speedup 1.0000x reference)
import jax
import jax.numpy as jnp
from jax import lax
from jax.experimental import pallas as pl
from jax.experimental.pallas import tpu as pltpu
from jax.experimental.pallas import tpu_sc as plsc

N = 10000
D = 128
E = 320000
NC = 2
NS = 16
NW = NC * NS
K = 128
CH = 80
EP = NW * K * CH
NP = 10240
RPT = NP // NS
BR = 512
NB = NP // BR

_mesh = plsc.VectorSubcoreMesh(
    core_axis_name="c", subcore_axis_name="s", num_cores=NC, num_subcores=NS
)


def _deg_body(dst_hbm, out_hbm, ones_v, idx_v, zer_v, acc_sh):
    c = lax.axis_index("c")
    s = lax.axis_index("s")
    w = c * NS + s
    for i in range(K // 16):
        ones_v[pl.ds(i * 16, 16)] = jnp.full((16,), 1.0, jnp.float32)
    for i in range(RPT // 16):
        zer_v[pl.ds(i * 16, 16)] = jnp.zeros((16,), jnp.float32)
    r0 = s * RPT
    pltpu.sync_copy(zer_v, acc_sh.at[pl.ds(r0, RPT)])
    pltpu.sync_copy(dst_hbm.at[w], idx_v)
    plsc.subcore_barrier()
    def body(ch, carry):
        pltpu.sync_copy(ones_v, acc_sh.at[idx_v.at[ch]], add=True)
        return carry
    lax.fori_loop(0, CH, body, 0)
    plsc.subcore_barrier()
    pltpu.sync_copy(acc_sh.at[pl.ds(r0, RPT)], out_hbm.at[pl.ds(c * NP + r0, RPT)])


_deg = pl.kernel(
    _deg_body,
    out_type=jax.ShapeDtypeStruct((NC * NP,), jnp.float32),
    mesh=_mesh,
    scratch_types=[
        pltpu.VMEM((K,), jnp.float32),
        pltpu.VMEM((CH, K), jnp.int32),
        pltpu.VMEM((RPT,), jnp.float32),
        pltpu.VMEM_SHARED((NP,), jnp.float32),
    ],
)


def _spmm_body(g_hbm, src_hbm, dst_hbm, zeros_hbm, out_hbm,
               dst_v, src0, src1, rows0, rows1, acc_sh,
               semi0, semi1, semg0, semg1):
    c = lax.axis_index("c")
    s = lax.axis_index("s")
    w = c * NS + s
    r0 = s * RPT
    @pl.when(c == 0)
    def _():
        pltpu.sync_copy(g_hbm.at[pl.ds(r0, RPT)], acc_sh.at[pl.ds(r0, RPT)])
    @pl.when(c != 0)
    def _():
        pltpu.sync_copy(zeros_hbm.at[pl.ds(r0, RPT)], acc_sh.at[pl.ds(r0, RPT)])
    pltpu.sync_copy(dst_hbm.at[w], dst_v)
    plsc.subcore_barrier()

    def idxload(ch, buf, sem):
        pltpu.async_copy(src_hbm.at[w, ch], buf, sem)

    def idxwait(ch, buf, sem):
        pltpu.make_async_copy(src_hbm.at[w, ch], buf, sem).wait()

    def gather(buf_idx, buf, sem):
        pltpu.async_copy(g_hbm.at[buf_idx], buf, sem)

    def gatherwait(buf_idx, buf, sem):
        pltpu.make_async_copy(g_hbm.at[buf_idx], buf, sem).wait()

    idxload(0, src0, semi0)
    idxload(1, src1, semi1)
    idxwait(0, src0, semi0)
    gather(src0, rows0, semg0)
    idxwait(1, src1, semi1)
    gather(src1, rows1, semg1)

    def half(ch, idx_b, rows_b, semi_b, semg_b):
        gatherwait(idx_b, rows_b, semg_b)
        @pl.when(ch + 2 < CH)
        def _():
            idxload(ch + 2, idx_b, semi_b)
        pltpu.sync_copy(rows_b, acc_sh.at[dst_v.at[ch]], add=True)
        @pl.when(ch + 2 < CH)
        def _():
            idxwait(ch + 2, idx_b, semi_b)
            gather(idx_b, rows_b, semg_b)

    def body(i, carry):
        ch0 = 2 * i
        half(ch0, src0, rows0, semi0, semg0)
        half(ch0 + 1, src1, rows1, semi1, semg1)
        return carry

    lax.fori_loop(0, CH // 2, body, 0)
    plsc.subcore_barrier()
    pltpu.sync_copy(acc_sh.at[pl.ds(r0, RPT)],
                    out_hbm.at[pl.ds(c * NP + r0, RPT)])


_spmm = pl.kernel(
    _spmm_body,
    out_type=jax.ShapeDtypeStruct((NC * NP, D), jnp.float32),
    mesh=_mesh,
    scratch_types=[
        pltpu.VMEM((CH, K), jnp.int32),
        pltpu.VMEM((K,), jnp.int32),
        pltpu.VMEM((K,), jnp.int32),
        pltpu.VMEM((K, D), jnp.float32),
        pltpu.VMEM((K, D), jnp.float32),
        pltpu.VMEM_SHARED((NP, D), jnp.float32),
        pltpu.SemaphoreType.DMA,
        pltpu.SemaphoreType.DMA,
        pltpu.SemaphoreType.DMA,
        pltpu.SemaphoreType.DMA,
    ],
)


def _dots_body(h_hbm, src_hbm, dst_hbm, out_hbm,
               src_v, dst_v, hs0, hd0, hs1, hd1, hs2, hd2,
               res0, res1, res2, tb_v,
               semg0, semg1, semg2, semr0, semr1, semr2):
    c = lax.axis_index("c")
    s = lax.axis_index("s")
    w = c * NS + s
    pltpu.sync_copy(src_hbm.at[w], src_v)
    pltpu.sync_copy(dst_hbm.at[w], dst_v)
    lanes = lax.iota(jnp.int32, 16)
    bufs = ((hs0, hd0, semg0), (hs1, hd1, semg1), (hs2, hd2, semg2))
    res_bufs = (res0, res1, res2)
    semr = (semr0, semr1, semr2)

    def gathers(ch, b):
        hs, hd, sem = bufs[b]
        pltpu.async_copy(h_hbm.at[src_v.at[ch]], hs, sem)
        pltpu.async_copy(h_hbm.at[dst_v.at[ch]], hd, sem)

    def wait(ch, b):
        hs, hd, sem = bufs[b]
        pltpu.make_async_copy(h_hbm.at[src_v.at[ch]], hs, sem).wait()
        pltpu.make_async_copy(h_hbm.at[dst_v.at[ch]], hd, sem).wait()

    addr_base = lanes * 16

    def dots(ch, b):
        hs, hd, _ = bufs[b]
        res = res_bufs[b]
        @plsc.parallel_loop(0, K // 16, 1, unroll=1)
        def group(g):
            @plsc.parallel_loop(0, 16, 1, unroll=4)
            def _(u):
                e = g * 16 + u
                acc0 = hs[e, pl.ds(0, 16)] * hd[e, pl.ds(0, 16)]
                acc1 = hs[e, pl.ds(16, 16)] * hd[e, pl.ds(16, 16)]
                for k in range(2, D // 16, 2):
                    acc0 = acc0 + hs[e, pl.ds(k * 16, 16)] * hd[e, pl.ds(k * 16, 16)]
                    acc1 = acc1 + hs[e, pl.ds((k + 1) * 16, 16)] * hd[e, pl.ds((k + 1) * 16, 16)]
                tb_v[pl.ds(g * 256 + u * 16, 16)] = acc0 + acc1
            base = g * 256 + addr_base
            tot0 = plsc.load_gather(tb_v, [base])
            tot1 = plsc.load_gather(tb_v, [base + 1])
            for j in range(2, 16, 2):
                tot0 = tot0 + plsc.load_gather(tb_v, [base + j])
                tot1 = tot1 + plsc.load_gather(tb_v, [base + (j + 1)])
            res[pl.ds(g * 16, 16)] = tot0 + tot1

    def res_out(ch, b):
        pltpu.async_copy(res_bufs[b], out_hbm.at[w, pl.ds(ch * K, K)],
                         semr[b])

    def res_wait(ch, b):
        pltpu.make_async_copy(res_bufs[b],
                              out_hbm.at[w, pl.ds(ch * K, K)],
                              semr[b]).wait()

    gathers(0, 0)
    gathers(1, 1)
    gathers(2, 2)

    def step(ch, b):
        wait(ch, b)
        if isinstance(ch, int):
            if ch >= 3:
                res_wait(ch - 3, b)
            dots(ch, b)
            res_out(ch, b)
            if ch + 3 < CH:
                gathers(ch + 3, b)
        else:
            @pl.when(ch >= 3)
            def _():
                res_wait(ch - 3, b)
            dots(ch, b)
            res_out(ch, b)
            @pl.when(ch + 3 < CH)
            def _():
                gathers(ch + 3, b)

    def body(i, carry):
        ch0 = 3 * i
        step(ch0, 0)
        step(ch0 + 1, 1)
        step(ch0 + 2, 2)
        return carry

    lax.fori_loop(0, CH // 3, body, 0)
    for ch in range(CH - CH % 3, CH):
        step(ch, ch % 3)
    for ch in range(CH - 3, CH):
        res_wait(ch, ch % 3)


_dots = pl.kernel(
    _dots_body,
    out_type=jax.ShapeDtypeStruct((NW, CH * K), jnp.float32),
    mesh=_mesh,
    compiler_params=pltpu.CompilerParams(needs_layout_passes=False),
    scratch_types=[
        pltpu.VMEM((CH, K), jnp.int32),
        pltpu.VMEM((CH, K), jnp.int32),
        pltpu.VMEM((K, D), jnp.float32),
        pltpu.VMEM((K, D), jnp.float32),
        pltpu.VMEM((K, D), jnp.float32),
        pltpu.VMEM((K, D), jnp.float32),
        pltpu.VMEM((K, D), jnp.float32),
        pltpu.VMEM((K, D), jnp.float32),
        pltpu.VMEM((K,), jnp.float32),
        pltpu.VMEM((K,), jnp.float32),
        pltpu.VMEM((K,), jnp.float32),
        pltpu.VMEM((2048,), jnp.float32),
        pltpu.SemaphoreType.DMA,
        pltpu.SemaphoreType.DMA,
        pltpu.SemaphoreType.DMA,
        pltpu.SemaphoreType.DMA,
        pltpu.SemaphoreType.DMA,
        pltpu.SemaphoreType.DMA,
    ],
)


def _tc1_body(x_ref, w_ref, degp_ref, g_ref, dis_ref):
    deg = degp_ref[0] + degp_ref[1] + 1.0
    dis = lax.rsqrt(deg)
    h = jnp.dot(x_ref[...], w_ref[...], preferred_element_type=jnp.float32)
    g_ref[...] = h * dis
    dis_ref[...] = dis


_tc1 = pl.pallas_call(
    _tc1_body,
    grid=(NB,),
    in_specs=[
        pl.BlockSpec((BR, D), lambda i: (i, 0)),
        pl.BlockSpec((D, D), lambda i: (0, 0)),
        pl.BlockSpec((2, BR, 1), lambda i: (0, i, 0)),
    ],
    out_specs=[
        pl.BlockSpec((BR, D), lambda i: (i, 0)),
        pl.BlockSpec((BR, 1), lambda i: (i, 0)),
    ],
    out_shape=[
        jax.ShapeDtypeStruct((NP, D), jnp.float32),
        jax.ShapeDtypeStruct((NP, 1), jnp.float32),
    ],
)


def _tc2_body(s_ref, dis_ref, b1_ref, w2_ref, g2_ref):
    agg = s_ref[0] + s_ref[1]
    a1 = jnp.maximum(dis_ref[...] * agg + b1_ref[...], 0.0)
    g2_ref[...] = jnp.dot(a1, w2_ref[...],
                          preferred_element_type=jnp.float32) * dis_ref[...]


_tc2 = pl.pallas_call(
    _tc2_body,
    grid=(NB,),
    in_specs=[
        pl.BlockSpec((2, BR, D), lambda i: (0, i, 0)),
        pl.BlockSpec((BR, 1), lambda i: (i, 0)),
        pl.BlockSpec((1, D), lambda i: (0, 0)),
        pl.BlockSpec((D, D), lambda i: (0, 0)),
    ],
    out_specs=pl.BlockSpec((BR, D), lambda i: (i, 0)),
    out_shape=jax.ShapeDtypeStruct((NP, D), jnp.float32),
)


def _tc3_body(s_ref, dis_ref, b2_ref, h2_ref):
    h2_ref[...] = dis_ref[...] * (s_ref[0] + s_ref[1]) + b2_ref[...]


_tc3 = pl.pallas_call(
    _tc3_body,
    grid=(NB,),
    in_specs=[
        pl.BlockSpec((2, BR, D), lambda i: (0, i, 0)),
        pl.BlockSpec((BR, 1), lambda i: (i, 0)),
        pl.BlockSpec((1, D), lambda i: (0, 0)),
    ],
    out_specs=pl.BlockSpec((BR, D), lambda i: (i, 0)),
    out_shape=jax.ShapeDtypeStruct((NP, D), jnp.float32),
)


def kernel(x, edge_index, W1, b1, W2, b2):
    src = edge_index[0]
    dst = edge_index[1]
    padi = (N + (jnp.arange(EP - E, dtype=jnp.int32) % (NP - N)))
    srcp = jnp.concatenate([src, padi]).reshape(NW, CH, K)
    dstp = jnp.concatenate([dst, padi]).reshape(NW, CH, K)
    xp = jnp.pad(x, ((0, NP - N), (0, 0)))
    zeros_nd = jnp.zeros((NP, D), jnp.float32)

    degp = _deg(dstp).reshape(NC, NP, 1)
    g1, dis = _tc1(xp, W1, degp)
    s1 = _spmm(g1, srcp, dstp, zeros_nd).reshape(NC, NP, D)
    g2 = _tc2(s1, dis, b1.reshape(1, D), W2)
    s2 = _spmm(g2, srcp, dstp, zeros_nd).reshape(NC, NP, D)
    h2 = _tc3(s2, dis, b2.reshape(1, D))
    logits = _dots(h2, srcp, dstp)
    return logits.reshape(EP)[:E]


# speedup vs baseline: 1.0040x; 1.0040x over previous
import jax
import jax.numpy as jnp
from jax import lax
from jax.experimental import pallas as pl
from jax.experimental.pallas import tpu as pltpu
from jax.experimental.pallas import tpu_sc as plsc

N = 10000
D = 128
E = 320000
NC = 2
NS = 16
NW = NC * NS
K = 128
CH = 80
EP = NW * K * CH
NP = 10240
RPT = NP // NS
BR = 512
NB = NP // BR

_mesh = plsc.VectorSubcoreMesh(
    core_axis_name="c", subcore_axis_name="s", num_cores=NC, num_subcores=NS
)


def _deg_body(dst_hbm, out_hbm, ones_v, idx_v, zer_v, acc_sh):
    c = lax.axis_index("c")
    s = lax.axis_index("s")
    w = c * NS + s
    for i in range(K // 16):
        ones_v[pl.ds(i * 16, 16)] = jnp.full((16,), 1.0, jnp.float32)
    for i in range(RPT // 16):
        zer_v[pl.ds(i * 16, 16)] = jnp.zeros((16,), jnp.float32)
    r0 = s * RPT
    pltpu.sync_copy(zer_v, acc_sh.at[pl.ds(r0, RPT)])
    pltpu.sync_copy(dst_hbm.at[w], idx_v)
    plsc.subcore_barrier()
    def body(ch, carry):
        pltpu.sync_copy(ones_v, acc_sh.at[idx_v.at[ch]], add=True)
        return carry
    lax.fori_loop(0, CH, body, 0)
    plsc.subcore_barrier()
    pltpu.sync_copy(acc_sh.at[pl.ds(r0, RPT)], out_hbm.at[pl.ds(c * NP + r0, RPT)])


_deg = pl.kernel(
    _deg_body,
    out_type=jax.ShapeDtypeStruct((NC * NP,), jnp.float32),
    mesh=_mesh,
    scratch_types=[
        pltpu.VMEM((K,), jnp.float32),
        pltpu.VMEM((CH, K), jnp.int32),
        pltpu.VMEM((RPT,), jnp.float32),
        pltpu.VMEM_SHARED((NP,), jnp.float32),
    ],
)


def _spmm_body(g_hbm, src_hbm, dst_hbm, zeros_hbm, out_hbm,
               src0, src1, dstb0, dstb1, rows0, rows1, acc_sh,
               semis0, semis1, semid0, semid1, semg0, semg1,
               sem_s0, sem_s1):
    c = lax.axis_index("c")
    s = lax.axis_index("s")
    w = c * NS + s
    r0 = s * RPT
    @pl.when(c == 0)
    def _():
        pltpu.sync_copy(g_hbm.at[pl.ds(r0, RPT)], acc_sh.at[pl.ds(r0, RPT)])
    @pl.when(c != 0)
    def _():
        pltpu.sync_copy(zeros_hbm.at[pl.ds(r0, RPT)], acc_sh.at[pl.ds(r0, RPT)])
    plsc.subcore_barrier()

    def srcload(ch, buf, sem):
        pltpu.async_copy(src_hbm.at[w, ch], buf, sem)

    def srcwait(ch, buf, sem):
        pltpu.make_async_copy(src_hbm.at[w, ch], buf, sem).wait()

    def dstload(ch, buf, sem):
        pltpu.async_copy(dst_hbm.at[w, pl.ds(2 * ch, 2)], buf, sem)

    def dstwait(ch, buf, sem):
        pltpu.make_async_copy(dst_hbm.at[w, pl.ds(2 * ch, 2)], buf,
                              sem).wait()

    def gather(buf_idx, buf, sem):
        pltpu.async_copy(g_hbm.at[buf_idx], buf, sem)

    def gatherwait(buf_idx, buf, sem):
        pltpu.make_async_copy(g_hbm.at[buf_idx], buf, sem).wait()

    srcload(0, src0, semis0)
    dstload(0, dstb0, semid0)
    srcload(1, src1, semis1)
    dstload(1, dstb1, semid1)
    srcwait(0, src0, semis0)
    gather(src0, rows0, semg0)
    srcwait(1, src1, semis1)
    gather(src1, rows1, semg1)

    def half(ch, sbuf, dbuf, rows_b, semis_b, semid_b, semg_b):
        gatherwait(sbuf, rows_b, semg_b)
        @pl.when(ch + 2 < CH)
        def _():
            srcload(ch + 2, sbuf, semis_b)
        dstwait(ch, dbuf, semid_b)
        c0 = pltpu.async_copy(rows_b.at[pl.ds(0, K // 2)],
                              acc_sh.at[dbuf.at[0]], sem_s0, add=True)
        c1 = pltpu.async_copy(rows_b.at[pl.ds(K // 2, K // 2)],
                              acc_sh.at[dbuf.at[1]], sem_s1, add=True)
        c0.wait()
        c1.wait()
        @pl.when(ch + 2 < CH)
        def _():
            dstload(ch + 2, dbuf, semid_b)
            srcwait(ch + 2, sbuf, semis_b)
            gather(sbuf, rows_b, semg_b)

    def body(i, carry):
        ch0 = 2 * i
        half(ch0, src0, dstb0, rows0, semis0, semid0, semg0)
        half(ch0 + 1, src1, dstb1, rows1, semis1, semid1, semg1)
        return carry

    lax.fori_loop(0, CH // 2, body, 0)
    plsc.subcore_barrier()
    pltpu.sync_copy(acc_sh.at[pl.ds(r0, RPT)],
                    out_hbm.at[pl.ds(c * NP + r0, RPT)])


_spmm = pl.kernel(
    _spmm_body,
    out_type=jax.ShapeDtypeStruct((NC * NP, D), jnp.float32),
    mesh=_mesh,
    scratch_types=[
        pltpu.VMEM((K,), jnp.int32),
        pltpu.VMEM((K,), jnp.int32),
        pltpu.VMEM((2, K // 2), jnp.int32),
        pltpu.VMEM((2, K // 2), jnp.int32),
        pltpu.VMEM((K, D), jnp.float32),
        pltpu.VMEM((K, D), jnp.float32),
        pltpu.VMEM_SHARED((NP, D), jnp.float32),
        pltpu.SemaphoreType.DMA,
        pltpu.SemaphoreType.DMA,
        pltpu.SemaphoreType.DMA,
        pltpu.SemaphoreType.DMA,
        pltpu.SemaphoreType.DMA,
        pltpu.SemaphoreType.DMA,
        pltpu.SemaphoreType.DMA,
        pltpu.SemaphoreType.DMA,
    ],
)


def _dots_body(h_hbm, src_hbm, dst_hbm, out_hbm,
               src_v, dst_v, hs0, hd0, hs1, hd1, hs2, hd2,
               res0, res1, res2, tb_v,
               semg0, semg1, semg2, semr0, semr1, semr2):
    c = lax.axis_index("c")
    s = lax.axis_index("s")
    w = c * NS + s
    pltpu.sync_copy(src_hbm.at[w], src_v)
    pltpu.sync_copy(dst_hbm.at[w], dst_v)
    lanes = lax.iota(jnp.int32, 16)
    bufs = ((hs0, hd0, semg0), (hs1, hd1, semg1), (hs2, hd2, semg2))
    res_bufs = (res0, res1, res2)
    semr = (semr0, semr1, semr2)

    def gathers(ch, b):
        hs, hd, sem = bufs[b]
        pltpu.async_copy(h_hbm.at[src_v.at[ch]], hs, sem)
        pltpu.async_copy(h_hbm.at[dst_v.at[ch]], hd, sem)

    def wait(ch, b):
        hs, hd, sem = bufs[b]
        pltpu.make_async_copy(h_hbm.at[src_v.at[ch]], hs, sem).wait()
        pltpu.make_async_copy(h_hbm.at[dst_v.at[ch]], hd, sem).wait()

    addr_base = lanes * 16

    def dots(ch, b):
        hs, hd, _ = bufs[b]
        res = res_bufs[b]
        @plsc.parallel_loop(0, K // 16, 1, unroll=1)
        def group(g):
            @plsc.parallel_loop(0, 16, 1, unroll=4)
            def _(u):
                e = g * 16 + u
                acc0 = hs[e, pl.ds(0, 16)] * hd[e, pl.ds(0, 16)]
                acc1 = hs[e, pl.ds(16, 16)] * hd[e, pl.ds(16, 16)]
                for k in range(2, D // 16, 2):
                    acc0 = acc0 + hs[e, pl.ds(k * 16, 16)] * hd[e, pl.ds(k * 16, 16)]
                    acc1 = acc1 + hs[e, pl.ds((k + 1) * 16, 16)] * hd[e, pl.ds((k + 1) * 16, 16)]
                tb_v[pl.ds(g * 256 + u * 16, 16)] = acc0 + acc1
            base = g * 256 + addr_base
            tot0 = plsc.load_gather(tb_v, [base])
            tot1 = plsc.load_gather(tb_v, [base + 1])
            for j in range(2, 16, 2):
                tot0 = tot0 + plsc.load_gather(tb_v, [base + j])
                tot1 = tot1 + plsc.load_gather(tb_v, [base + (j + 1)])
            res[pl.ds(g * 16, 16)] = tot0 + tot1

    def res_out(ch, b):
        pltpu.async_copy(res_bufs[b], out_hbm.at[w, pl.ds(ch * K, K)],
                         semr[b])

    def res_wait(ch, b):
        pltpu.make_async_copy(res_bufs[b],
                              out_hbm.at[w, pl.ds(ch * K, K)],
                              semr[b]).wait()

    gathers(0, 0)
    gathers(1, 1)
    gathers(2, 2)

    def step(ch, b):
        wait(ch, b)
        if isinstance(ch, int):
            if ch >= 3:
                res_wait(ch - 3, b)
            dots(ch, b)
            res_out(ch, b)
            if ch + 3 < CH:
                gathers(ch + 3, b)
        else:
            @pl.when(ch >= 3)
            def _():
                res_wait(ch - 3, b)
            dots(ch, b)
            res_out(ch, b)
            @pl.when(ch + 3 < CH)
            def _():
                gathers(ch + 3, b)

    def body(i, carry):
        ch0 = 3 * i
        step(ch0, 0)
        step(ch0 + 1, 1)
        step(ch0 + 2, 2)
        return carry

    lax.fori_loop(0, CH // 3, body, 0)
    for ch in range(CH - CH % 3, CH):
        step(ch, ch % 3)
    for ch in range(CH - 3, CH):
        res_wait(ch, ch % 3)


_dots = pl.kernel(
    _dots_body,
    out_type=jax.ShapeDtypeStruct((NW, CH * K), jnp.float32),
    mesh=_mesh,
    compiler_params=pltpu.CompilerParams(needs_layout_passes=False),
    scratch_types=[
        pltpu.VMEM((CH, K), jnp.int32),
        pltpu.VMEM((CH, K), jnp.int32),
        pltpu.VMEM((K, D), jnp.float32),
        pltpu.VMEM((K, D), jnp.float32),
        pltpu.VMEM((K, D), jnp.float32),
        pltpu.VMEM((K, D), jnp.float32),
        pltpu.VMEM((K, D), jnp.float32),
        pltpu.VMEM((K, D), jnp.float32),
        pltpu.VMEM((K,), jnp.float32),
        pltpu.VMEM((K,), jnp.float32),
        pltpu.VMEM((K,), jnp.float32),
        pltpu.VMEM((2048,), jnp.float32),
        pltpu.SemaphoreType.DMA,
        pltpu.SemaphoreType.DMA,
        pltpu.SemaphoreType.DMA,
        pltpu.SemaphoreType.DMA,
        pltpu.SemaphoreType.DMA,
        pltpu.SemaphoreType.DMA,
    ],
)


def _tc1_body(x_ref, w_ref, degp_ref, g_ref, dis_ref):
    deg = degp_ref[0] + degp_ref[1] + 1.0
    dis = lax.rsqrt(deg)
    h = jnp.dot(x_ref[...], w_ref[...], preferred_element_type=jnp.float32)
    g_ref[...] = h * dis
    dis_ref[...] = dis


_tc1 = pl.pallas_call(
    _tc1_body,
    grid=(NB,),
    in_specs=[
        pl.BlockSpec((BR, D), lambda i: (i, 0)),
        pl.BlockSpec((D, D), lambda i: (0, 0)),
        pl.BlockSpec((2, BR, 1), lambda i: (0, i, 0)),
    ],
    out_specs=[
        pl.BlockSpec((BR, D), lambda i: (i, 0)),
        pl.BlockSpec((BR, 1), lambda i: (i, 0)),
    ],
    out_shape=[
        jax.ShapeDtypeStruct((NP, D), jnp.float32),
        jax.ShapeDtypeStruct((NP, 1), jnp.float32),
    ],
)


def _tc2_body(s_ref, dis_ref, b1_ref, w2_ref, g2_ref):
    agg = s_ref[0] + s_ref[1]
    a1 = jnp.maximum(dis_ref[...] * agg + b1_ref[...], 0.0)
    g2_ref[...] = jnp.dot(a1, w2_ref[...],
                          preferred_element_type=jnp.float32) * dis_ref[...]


_tc2 = pl.pallas_call(
    _tc2_body,
    grid=(NB,),
    in_specs=[
        pl.BlockSpec((2, BR, D), lambda i: (0, i, 0)),
        pl.BlockSpec((BR, 1), lambda i: (i, 0)),
        pl.BlockSpec((1, D), lambda i: (0, 0)),
        pl.BlockSpec((D, D), lambda i: (0, 0)),
    ],
    out_specs=pl.BlockSpec((BR, D), lambda i: (i, 0)),
    out_shape=jax.ShapeDtypeStruct((NP, D), jnp.float32),
)


def _tc3_body(s_ref, dis_ref, b2_ref, h2_ref):
    h2_ref[...] = dis_ref[...] * (s_ref[0] + s_ref[1]) + b2_ref[...]


_tc3 = pl.pallas_call(
    _tc3_body,
    grid=(NB,),
    in_specs=[
        pl.BlockSpec((2, BR, D), lambda i: (0, i, 0)),
        pl.BlockSpec((BR, 1), lambda i: (i, 0)),
        pl.BlockSpec((1, D), lambda i: (0, 0)),
    ],
    out_specs=pl.BlockSpec((BR, D), lambda i: (i, 0)),
    out_shape=jax.ShapeDtypeStruct((NP, D), jnp.float32),
)


def kernel(x, edge_index, W1, b1, W2, b2):
    src = edge_index[0]
    dst = edge_index[1]
    padi = (N + (jnp.arange(EP - E, dtype=jnp.int32) % (NP - N)))
    srcp = jnp.concatenate([src, padi]).reshape(NW, CH, K)
    dstp = jnp.concatenate([dst, padi]).reshape(NW, CH, K)
    dstp2 = dstp.reshape(NW, 2 * CH, K // 2)
    xp = jnp.pad(x, ((0, NP - N), (0, 0)))
    zeros_nd = jnp.zeros((NP, D), jnp.float32)

    degp = _deg(dstp).reshape(NC, NP, 1)
    g1, dis = _tc1(xp, W1, degp)
    s1 = _spmm(g1, srcp, dstp2, zeros_nd).reshape(NC, NP, D)
    g2 = _tc2(s1, dis, b1.reshape(1, D), W2)
    s2 = _spmm(g2, srcp, dstp2, zeros_nd).reshape(NC, NP, D)
    h2 = _tc3(s2, dis, b2.reshape(1, D))
    logits = _dots(h2, srcp, dstp)
    return logits.reshape(EP)[:E]


# speedup vs baseline: 1.0651x; 1.0608x over previous
import jax
import jax.numpy as jnp
from jax import lax
from jax.experimental import pallas as pl
from jax.experimental.pallas import tpu as pltpu
from jax.experimental.pallas import tpu_sc as plsc

N = 10000
D = 128
E = 320000
NC = 2
NS = 16
NW = NC * NS
K = 128
CH = 80
EP = NW * K * CH
NP = 10240
RPT = NP // NS
BR = 2048
NB = NP // BR

_mesh = plsc.VectorSubcoreMesh(
    core_axis_name="c", subcore_axis_name="s", num_cores=NC, num_subcores=NS
)

_PADI = N + (jnp.arange(EP - E, dtype=jnp.int32) % (NP - N))


def _deg_body(dst_hbm, out_hbm, ones_v, idx_v, zer_v, acc_sh):
    c = lax.axis_index("c")
    s = lax.axis_index("s")
    w = c * NS + s
    for i in range(K // 16):
        ones_v[pl.ds(i * 16, 16)] = jnp.full((16,), 1.0, jnp.float32)
    for i in range(RPT // 16):
        zer_v[pl.ds(i * 16, 16)] = jnp.zeros((16,), jnp.float32)
    r0 = s * RPT
    pltpu.sync_copy(zer_v, acc_sh.at[pl.ds(r0, RPT)])
    pltpu.sync_copy(dst_hbm.at[w], idx_v)
    plsc.subcore_barrier()
    def body(ch, carry):
        pltpu.sync_copy(ones_v, acc_sh.at[idx_v.at[ch]], add=True)
        return carry
    lax.fori_loop(0, CH, body, 0)
    plsc.subcore_barrier()
    pltpu.sync_copy(acc_sh.at[pl.ds(r0, RPT)], out_hbm.at[pl.ds(c * NP + r0, RPT)])


_deg = pl.kernel(
    _deg_body,
    out_type=jax.ShapeDtypeStruct((NC * NP,), jnp.float32),
    mesh=_mesh,
    scratch_types=[
        pltpu.VMEM((K,), jnp.float32),
        pltpu.VMEM((CH, K), jnp.int32),
        pltpu.VMEM((RPT,), jnp.float32),
        pltpu.VMEM_SHARED((NP,), jnp.float32),
    ],
)


def _spmm_body(g_hbm, src_hbm, dst_hbm, out_hbm,
               src0, src1, dstb0, dstb1, rows0, rows1, acc_sh,
               semis0, semis1, semid0, semid1, semg0, semg1,
               sem_s0, sem_s1):
    c = lax.axis_index("c")
    s = lax.axis_index("s")
    w = c * NS + s
    r0 = s * RPT
    @pl.when(c == 0)
    def _():
        pltpu.sync_copy(g_hbm.at[pl.ds(r0, RPT)], acc_sh.at[pl.ds(r0, RPT)])
    @pl.when(c != 0)
    def _():
        def zrow(r, carry):
            for kk in range(D // 16):
                rows0[r, pl.ds(kk * 16, 16)] = jnp.zeros((16,), jnp.float32)
            return carry
        lax.fori_loop(0, K, zrow, 0)
        for blk in range(RPT // K):
            pltpu.sync_copy(rows0, acc_sh.at[pl.ds(r0 + blk * K, K)])
    plsc.subcore_barrier()

    def srcload(ch, buf, sem):
        pltpu.async_copy(src_hbm.at[w, ch], buf, sem)

    def srcwait(ch, buf, sem):
        pltpu.make_async_copy(src_hbm.at[w, ch], buf, sem).wait()

    def dstload(ch, buf, sem):
        pltpu.async_copy(dst_hbm.at[w, pl.ds(2 * ch, 2)], buf, sem)

    def dstwait(ch, buf, sem):
        pltpu.make_async_copy(dst_hbm.at[w, pl.ds(2 * ch, 2)], buf,
                              sem).wait()

    def gather(buf_idx, buf, sem):
        pltpu.async_copy(g_hbm.at[buf_idx], buf, sem)

    def gatherwait(buf_idx, buf, sem):
        pltpu.make_async_copy(g_hbm.at[buf_idx], buf, sem).wait()

    srcload(0, src0, semis0)
    dstload(0, dstb0, semid0)
    srcload(1, src1, semis1)
    dstload(1, dstb1, semid1)
    srcwait(0, src0, semis0)
    gather(src0, rows0, semg0)
    srcwait(1, src1, semis1)
    gather(src1, rows1, semg1)

    def half(ch, sbuf, dbuf, rows_b, semis_b, semid_b, semg_b):
        gatherwait(sbuf, rows_b, semg_b)
        @pl.when(ch + 2 < CH)
        def _():
            srcload(ch + 2, sbuf, semis_b)
        dstwait(ch, dbuf, semid_b)
        c0 = pltpu.async_copy(rows_b.at[pl.ds(0, K // 2)],
                              acc_sh.at[dbuf.at[0]], sem_s0, add=True)
        c1 = pltpu.async_copy(rows_b.at[pl.ds(K // 2, K // 2)],
                              acc_sh.at[dbuf.at[1]], sem_s1, add=True)
        c0.wait()
        c1.wait()
        @pl.when(ch + 2 < CH)
        def _():
            dstload(ch + 2, dbuf, semid_b)
            srcwait(ch + 2, sbuf, semis_b)
            gather(sbuf, rows_b, semg_b)

    def body(i, carry):
        ch0 = 2 * i
        half(ch0, src0, dstb0, rows0, semis0, semid0, semg0)
        half(ch0 + 1, src1, dstb1, rows1, semis1, semid1, semg1)
        return carry

    lax.fori_loop(0, CH // 2, body, 0)
    plsc.subcore_barrier()
    pltpu.sync_copy(acc_sh.at[pl.ds(r0, RPT)],
                    out_hbm.at[pl.ds(c * NP + r0, RPT)])


_spmm = pl.kernel(
    _spmm_body,
    out_type=jax.ShapeDtypeStruct((NC * NP, D), jnp.float32),
    mesh=_mesh,
    scratch_types=[
        pltpu.VMEM((K,), jnp.int32),
        pltpu.VMEM((K,), jnp.int32),
        pltpu.VMEM((2, K // 2), jnp.int32),
        pltpu.VMEM((2, K // 2), jnp.int32),
        pltpu.VMEM((K, D), jnp.float32),
        pltpu.VMEM((K, D), jnp.float32),
        pltpu.VMEM_SHARED((NP, D), jnp.float32),
        pltpu.SemaphoreType.DMA,
        pltpu.SemaphoreType.DMA,
        pltpu.SemaphoreType.DMA,
        pltpu.SemaphoreType.DMA,
        pltpu.SemaphoreType.DMA,
        pltpu.SemaphoreType.DMA,
        pltpu.SemaphoreType.DMA,
        pltpu.SemaphoreType.DMA,
    ],
)


def _dots_body(h_hbm, src_hbm, dst_hbm, out_hbm,
               src_v, dst_v, hs0, hd0, hs1, hd1, hs2, hd2,
               res0, res1, res2, tb_v,
               semg0, semg1, semg2, semr0, semr1, semr2):
    c = lax.axis_index("c")
    s = lax.axis_index("s")
    w = c * NS + s
    pltpu.sync_copy(src_hbm.at[w], src_v)
    pltpu.sync_copy(dst_hbm.at[w], dst_v)
    lanes = lax.iota(jnp.int32, 16)
    bufs = ((hs0, hd0, semg0), (hs1, hd1, semg1), (hs2, hd2, semg2))
    res_bufs = (res0, res1, res2)
    semr = (semr0, semr1, semr2)

    def gathers(ch, b):
        hs, hd, sem = bufs[b]
        pltpu.async_copy(h_hbm.at[src_v.at[ch]], hs, sem)
        pltpu.async_copy(h_hbm.at[dst_v.at[ch]], hd, sem)

    def wait(ch, b):
        hs, hd, sem = bufs[b]
        pltpu.make_async_copy(h_hbm.at[src_v.at[ch]], hs, sem).wait()
        pltpu.make_async_copy(h_hbm.at[dst_v.at[ch]], hd, sem).wait()

    addr_base = lanes * 16

    def dots(ch, b):
        hs, hd, _ = bufs[b]
        res = res_bufs[b]
        @plsc.parallel_loop(0, K // 16, 1, unroll=1)
        def group(g):
            @plsc.parallel_loop(0, 16, 1, unroll=4)
            def _(u):
                e = g * 16 + u
                acc0 = hs[e, pl.ds(0, 16)] * hd[e, pl.ds(0, 16)]
                acc1 = hs[e, pl.ds(16, 16)] * hd[e, pl.ds(16, 16)]
                for k in range(2, D // 16, 2):
                    acc0 = acc0 + hs[e, pl.ds(k * 16, 16)] * hd[e, pl.ds(k * 16, 16)]
                    acc1 = acc1 + hs[e, pl.ds((k + 1) * 16, 16)] * hd[e, pl.ds((k + 1) * 16, 16)]
                tb_v[pl.ds(g * 256 + u * 16, 16)] = acc0 + acc1
            base = g * 256 + addr_base
            tot0 = plsc.load_gather(tb_v, [base])
            tot1 = plsc.load_gather(tb_v, [base + 1])
            for j in range(2, 16, 2):
                tot0 = tot0 + plsc.load_gather(tb_v, [base + j])
                tot1 = tot1 + plsc.load_gather(tb_v, [base + (j + 1)])
            res[pl.ds(g * 16, 16)] = tot0 + tot1

    def res_out(ch, b):
        pltpu.async_copy(res_bufs[b], out_hbm.at[w, pl.ds(ch * K, K)],
                         semr[b])

    def res_wait(ch, b):
        pltpu.make_async_copy(res_bufs[b],
                              out_hbm.at[w, pl.ds(ch * K, K)],
                              semr[b]).wait()

    gathers(0, 0)
    gathers(1, 1)
    gathers(2, 2)

    def step(ch, b):
        wait(ch, b)
        if isinstance(ch, int):
            if ch >= 3:
                res_wait(ch - 3, b)
            dots(ch, b)
            res_out(ch, b)
            if ch + 3 < CH:
                gathers(ch + 3, b)
        else:
            @pl.when(ch >= 3)
            def _():
                res_wait(ch - 3, b)
            dots(ch, b)
            res_out(ch, b)
            @pl.when(ch + 3 < CH)
            def _():
                gathers(ch + 3, b)

    def body(i, carry):
        ch0 = 3 * i
        step(ch0, 0)
        step(ch0 + 1, 1)
        step(ch0 + 2, 2)
        return carry

    lax.fori_loop(0, CH // 3, body, 0)
    for ch in range(CH - CH % 3, CH):
        step(ch, ch % 3)
    for ch in range(CH - 3, CH):
        res_wait(ch, ch % 3)


_dots = pl.kernel(
    _dots_body,
    out_type=jax.ShapeDtypeStruct((NW, CH * K), jnp.float32),
    mesh=_mesh,
    compiler_params=pltpu.CompilerParams(needs_layout_passes=False),
    scratch_types=[
        pltpu.VMEM((CH, K), jnp.int32),
        pltpu.VMEM((CH, K), jnp.int32),
        pltpu.VMEM((K, D), jnp.float32),
        pltpu.VMEM((K, D), jnp.float32),
        pltpu.VMEM((K, D), jnp.float32),
        pltpu.VMEM((K, D), jnp.float32),
        pltpu.VMEM((K, D), jnp.float32),
        pltpu.VMEM((K, D), jnp.float32),
        pltpu.VMEM((K,), jnp.float32),
        pltpu.VMEM((K,), jnp.float32),
        pltpu.VMEM((K,), jnp.float32),
        pltpu.VMEM((2048,), jnp.float32),
        pltpu.SemaphoreType.DMA,
        pltpu.SemaphoreType.DMA,
        pltpu.SemaphoreType.DMA,
        pltpu.SemaphoreType.DMA,
        pltpu.SemaphoreType.DMA,
        pltpu.SemaphoreType.DMA,
    ],
)


def _tc1_body(x_ref, w_ref, degp_ref, g_ref, dis_ref):
    deg = degp_ref[0] + degp_ref[1] + 1.0
    dis = lax.rsqrt(deg)
    h = jnp.dot(x_ref[...], w_ref[...], preferred_element_type=jnp.float32)
    g_ref[...] = h * dis
    dis_ref[...] = dis


_tc1 = pl.pallas_call(
    _tc1_body,
    grid=(NB,),
    in_specs=[
        pl.BlockSpec((BR, D), lambda i: (i, 0)),
        pl.BlockSpec((D, D), lambda i: (0, 0)),
        pl.BlockSpec((2, BR, 1), lambda i: (0, i, 0)),
    ],
    out_specs=[
        pl.BlockSpec((BR, D), lambda i: (i, 0)),
        pl.BlockSpec((BR, 1), lambda i: (i, 0)),
    ],
    out_shape=[
        jax.ShapeDtypeStruct((NP, D), jnp.float32),
        jax.ShapeDtypeStruct((NP, 1), jnp.float32),
    ],
)


def _tc2_body(s_ref, dis_ref, b1_ref, w2_ref, g2_ref):
    agg = s_ref[0] + s_ref[1]
    a1 = jnp.maximum(dis_ref[...] * agg + b1_ref[...], 0.0)
    g2_ref[...] = jnp.dot(a1, w2_ref[...],
                          preferred_element_type=jnp.float32) * dis_ref[...]


_tc2 = pl.pallas_call(
    _tc2_body,
    grid=(NB,),
    in_specs=[
        pl.BlockSpec((2, BR, D), lambda i: (0, i, 0)),
        pl.BlockSpec((BR, 1), lambda i: (i, 0)),
        pl.BlockSpec((1, D), lambda i: (0, 0)),
        pl.BlockSpec((D, D), lambda i: (0, 0)),
    ],
    out_specs=pl.BlockSpec((BR, D), lambda i: (i, 0)),
    out_shape=jax.ShapeDtypeStruct((NP, D), jnp.float32),
)


def _tc3_body(s_ref, dis_ref, b2_ref, h2_ref):
    h2_ref[...] = dis_ref[...] * (s_ref[0] + s_ref[1]) + b2_ref[...]


_tc3 = pl.pallas_call(
    _tc3_body,
    grid=(NB,),
    in_specs=[
        pl.BlockSpec((2, BR, D), lambda i: (0, i, 0)),
        pl.BlockSpec((BR, 1), lambda i: (i, 0)),
        pl.BlockSpec((1, D), lambda i: (0, 0)),
    ],
    out_specs=pl.BlockSpec((BR, D), lambda i: (i, 0)),
    out_shape=jax.ShapeDtypeStruct((NP, D), jnp.float32),
)


def kernel(x, edge_index, W1, b1, W2, b2):
    src = edge_index[0]
    dst = edge_index[1]
    srcp = jnp.concatenate([src, _PADI]).reshape(NW, CH, K)
    dstp = jnp.concatenate([dst, _PADI]).reshape(NW, CH, K)
    dstp2 = dstp.reshape(NW, 2 * CH, K // 2)
    xp = jnp.pad(x, ((0, NP - N), (0, 0)))

    degp = _deg(dstp).reshape(NC, NP, 1)
    g1, dis = _tc1(xp, W1, degp)
    s1 = _spmm(g1, srcp, dstp2).reshape(NC, NP, D)
    g2 = _tc2(s1, dis, b1.reshape(1, D), W2)
    s2 = _spmm(g2, srcp, dstp2).reshape(NC, NP, D)
    h2 = _tc3(s2, dis, b2.reshape(1, D))
    logits = _dots(h2, srcp, dstp)
    return logits.reshape(EP)[:E]
